# baseline (speedup 1.0000x reference)
import functools

import jax
import jax.numpy as jnp
from jax import lax
from jax.experimental import pallas as pl
from jax.experimental.pallas import tpu as pltpu
from jax.experimental.pallas import tpu_sc as plsc

NC = 2
NS = 16
NW = NC * NS
CHUNK = 120
NBUF = 3
LAG = 2
ISN = 6
DDEPTH = 3
BLK = 2048

_mesh = functools.partial(
    plsc.VectorSubcoreMesh, core_axis_name="c", subcore_axis_name="s",
    num_cores=NC, num_subcores=NS)


def _sc_degrees(ei, zeros1, n_pad, nchunk):
  slab_n = n_pad // NS

  @functools.partial(
      pl.kernel,
      out_type=jax.ShapeDtypeStruct((NC, 2, n_pad), jnp.float32),
      mesh=_mesh(),
      scratch_types=[
          pltpu.VMEM((2, nchunk, CHUNK), jnp.int32),
          pltpu.VMEM((CHUNK,), jnp.float32),
          pltpu.VMEM_SHARED((n_pad,), jnp.float32),
          pltpu.VMEM_SHARED((n_pad,), jnp.float32),
          pltpu.SemaphoreType.DMA((DDEPTH + 1,)),
          pltpu.SemaphoreType.DMA((DDEPTH + 1,)),
      ],
  )
  def k(ei_hbm, z_hbm, deg_hbm, idx_v, ones_v, dego_sh, degi_sh, osem, isem):
    cid = lax.axis_index("c")
    sid = lax.axis_index("s")
    w = cid * NS + sid
    pltpu.sync_copy(ei_hbm.at[0, w], idx_v.at[0])
    pltpu.sync_copy(ei_hbm.at[1, w], idx_v.at[1])
    for t in range(CHUNK // 16):
      ones_v[pl.ds(t * 16, 16)] = jnp.ones((16,), jnp.float32)
    slab = pl.ds(sid * slab_n, slab_n)
    pltpu.sync_copy(z_hbm.at[slab], dego_sh.at[slab])
    pltpu.sync_copy(z_hbm.at[slab], degi_sh.at[slab])
    plsc.subcore_barrier()

    def fire(c):
      cb = lax.rem(c, DDEPTH + 1)
      pltpu.async_copy(ones_v, dego_sh.at[idx_v.at[0, c]], osem.at[cb],
                       add=True)
      pltpu.async_copy(ones_v, degi_sh.at[idx_v.at[1, c]], isem.at[cb],
                       add=True)

    def drain(c):
      cb = lax.rem(c, DDEPTH + 1)
      pltpu.make_async_copy(
          ones_v, dego_sh.at[idx_v.at[0, c]], osem.at[cb]).wait()
      pltpu.make_async_copy(
          ones_v, degi_sh.at[idx_v.at[1, c]], isem.at[cb]).wait()

    for c in range(DDEPTH):
      fire(c)

    def body(j, carry):
      @pl.when(j + DDEPTH < nchunk)
      def _():
        fire(j + DDEPTH)

      drain(j)
      return carry

    lax.fori_loop(0, nchunk, body, 0)
    plsc.subcore_barrier()
    pltpu.sync_copy(dego_sh.at[slab], deg_hbm.at[cid, 0, slab])
    pltpu.sync_copy(degi_sh.at[slab], deg_hbm.at[cid, 1, slab])

  return k(ei, zeros1)


def _sc_aggregate(ei, h_scaled, zeros2, n_acc, nchunk, d):
  slab_n = n_acc // NS
  ilead = DDEPTH + LAG

  @functools.partial(
      pl.kernel,
      out_type=jax.ShapeDtypeStruct((NC, n_acc, d), jnp.float32),
      mesh=_mesh(),
      scratch_types=[
          pltpu.VMEM((ISN, CHUNK), jnp.int32),
          pltpu.VMEM((ISN, CHUNK), jnp.int32),
          pltpu.VMEM((NBUF, CHUNK, d), jnp.float32),
          pltpu.VMEM_SHARED((n_acc, d), jnp.float32),
          pltpu.SemaphoreType.DMA((NBUF,)),
          pltpu.SemaphoreType.DMA((NBUF,)),
          pltpu.SemaphoreType.DMA((ISN,)),
          pltpu.SemaphoreType.DMA((ISN,)),
      ],
  )
  def k(ei_hbm, h_hbm, z_hbm, agg_hbm,
        src_v, dst_v, rows_v, agg_sh, gsem, ssem, xsem, ysem):
    cid = lax.axis_index("c")
    sid = lax.axis_index("s")
    w = cid * NS + sid
    slab = pl.ds(sid * slab_n, slab_n)
    pltpu.sync_copy(z_hbm.at[slab], agg_sh.at[slab])
    plsc.subcore_barrier()

    def fire_src(r, rb):
      pltpu.async_copy(ei_hbm.at[0, w, r], src_v.at[rb], xsem.at[rb])

    def fire_dst(r, rb):
      pltpu.async_copy(ei_hbm.at[1, w, r], dst_v.at[rb], ysem.at[rb])

    def fire_gather(r, rb, bb):
      pltpu.make_async_copy(
          ei_hbm.at[0, w, r], src_v.at[rb], xsem.at[rb]).wait()
      pltpu.async_copy(h_hbm.at[src_v.at[rb]], rows_v.at[bb], gsem.at[bb])

    for r in range(min(ilead, nchunk)):
      fire_src(r, r % ISN)
    for r in range(min(DDEPTH, nchunk)):
      fire_dst(r, r % ISN)
    for bi in range(min(NBUF, nchunk)):
      fire_gather(bi, bi % ISN, bi)

    def body(j, carry):
      jb = lax.rem(j, NBUF)
      js = lax.rem(j, ISN)

      @pl.when(j + ilead < nchunk)
      def _():
        fire_src(j + ilead, lax.rem(j + ilead, ISN))

      @pl.when(j + DDEPTH < nchunk)
      def _():
        fire_dst(j + DDEPTH, lax.rem(j + DDEPTH, ISN))

      pltpu.make_async_copy(
          h_hbm.at[src_v.at[js]], rows_v.at[jb], gsem.at[jb]).wait()
      pltpu.make_async_copy(
          ei_hbm.at[1, w, j], dst_v.at[js], ysem.at[js]).wait()
      pltpu.async_copy(
          rows_v.at[jb], agg_sh.at[dst_v.at[js]], ssem.at[jb], add=True)

      @pl.when(j >= LAG)
      def _():
        jp = j - LAG
        pb = lax.rem(jp, NBUF)
        pltpu.make_async_copy(
            rows_v.at[pb], agg_sh.at[dst_v.at[lax.rem(jp, ISN)]],
            ssem.at[pb]).wait()

        @pl.when(jp + NBUF < nchunk)
        def _():
          jn = jp + NBUF
          fire_gather(jn, lax.rem(jn, ISN), pb)

      return carry

    lax.fori_loop(0, nchunk, body, 0)
    for t in range(max(nchunk - LAG, 0), nchunk):
      pltpu.make_async_copy(
          rows_v.at[t % NBUF], agg_sh.at[dst_v.at[t % ISN]],
          ssem.at[t % NBUF]).wait()
    plsc.subcore_barrier()
    pltpu.sync_copy(agg_sh.at[slab], agg_hbm.at[cid, slab])

  return k(ei, h_scaled, zeros2)


def _tc_matmul_scale(x_pad, w, degp, n_pad, d):
  grid = n_pad // BLK

  def body(x_ref, w_ref, deg_ref, o_ref):
    deg = deg_ref[0, 0] + deg_ref[1, 0]
    norm = lax.rsqrt(jnp.maximum(deg, 1.0))
    h = jnp.dot(x_ref[...], w_ref[...], preferred_element_type=jnp.float32)
    o_ref[...] = h * norm

  return pl.pallas_call(
      body,
      grid=(grid,),
      in_specs=[
          pl.BlockSpec((BLK, d), lambda i: (i, 0)),
          pl.BlockSpec((d, d), lambda i: (0, 0)),
          pl.BlockSpec((NC, 2, BLK, 1), lambda i: (0, 0, i, 0)),
      ],
      out_specs=pl.BlockSpec((BLK, d), lambda i: (i, 0)),
      out_shape=jax.ShapeDtypeStruct((n_pad, d), jnp.float32),
  )(x_pad, w, degp)


def _tc_finalize(aggp, degp, b2, n_acc, d):
  blk2 = n_acc // 4
  assert blk2 % 8 == 0
  grid = n_acc // blk2

  def body(agg_ref, deg_ref, b_ref, act_ref, clone_ref):
    agg = agg_ref[0] + agg_ref[1]
    deg = deg_ref[0, 1] + deg_ref[1, 1]
    norm = lax.rsqrt(jnp.maximum(deg, 1.0))
    out = agg * norm + b_ref[...]
    act = jnp.maximum(out, 0.0)
    act_ref[...] = act
    clone_ref[...] = jnp.where(act >= 0.5, 1.0, 0.0).astype(jnp.float32)

  return pl.pallas_call(
      body,
      grid=(grid,),
      in_specs=[
          pl.BlockSpec((NC, blk2, d), lambda i: (0, i, 0)),
          pl.BlockSpec((NC, 2, blk2, 1), lambda i: (0, 0, i, 0)),
          pl.BlockSpec((1, d), lambda i: (0, 0)),
      ],
      out_specs=[
          pl.BlockSpec((blk2, d), lambda i: (i, 0)),
          pl.BlockSpec((blk2, d), lambda i: (i, 0)),
      ],
      out_shape=[
          jax.ShapeDtypeStruct((n_acc, d), jnp.float32),
          jax.ShapeDtypeStruct((n_acc, d), jnp.float32),
      ],
  )(aggp, degp, b2)


def kernel(in_feat, edge_index, W, b):
  n, d = in_feat.shape
  e = edge_index.shape[1]
  n_pad = ((n + BLK - 1) // BLK) * BLK
  n_acc = 128 * (-(-(n + 1) // 128))
  nchunk = -(-e // (NW * CHUNK))
  pad_len = NW * CHUNK * nchunk - e
  pad_vals = n + (jnp.arange(pad_len, dtype=jnp.int32) % (n_acc - n))
  ei = jnp.concatenate(
      [edge_index.astype(jnp.int32),
       jnp.broadcast_to(pad_vals, (2, pad_len))],
      axis=1).reshape(2, nchunk, NW, CHUNK).transpose(0, 2, 1, 3)

  zeros1 = jnp.zeros((n_pad,), jnp.float32)
  zeros2 = jnp.zeros((n_acc, d), jnp.float32)
  x_pad = jnp.pad(in_feat, ((0, n_pad - n), (0, 0)))

  degp = _sc_degrees(ei, zeros1, n_pad, nchunk)
  degp4 = degp.reshape(NC, 2, n_pad, 1)
  h_scaled = _tc_matmul_scale(x_pad, W, degp4, n_pad, d)
  aggp = _sc_aggregate(ei, h_scaled, zeros2, n_acc, nchunk, d)
  h_act, h_clone = _tc_finalize(aggp, degp4[:, :, :n_acc], b.reshape(1, d),
                                n_acc, d)
  return (h_act[:n], h_clone[:n])


# speedup vs baseline: 1.0417x; 1.0417x over previous
import functools

import jax
import jax.numpy as jnp
from jax import lax
from jax.experimental import pallas as pl
from jax.experimental.pallas import tpu as pltpu
from jax.experimental.pallas import tpu_sc as plsc

NC = 2
NS = 16
NW = NC * NS
CHUNK = 128
NBUF = 2
ISN = 6
DDEPTH = 3
BLK = 2048

_mesh = functools.partial(
    plsc.VectorSubcoreMesh, core_axis_name="c", subcore_axis_name="s",
    num_cores=NC, num_subcores=NS)


def _sc_degrees(ei, zeros1, n_pad, nchunk):
  slab_n = n_pad // NS

  @functools.partial(
      pl.kernel,
      out_type=jax.ShapeDtypeStruct((NC, 2, n_pad), jnp.float32),
      mesh=_mesh(),
      scratch_types=[
          pltpu.VMEM((2, nchunk, CHUNK), jnp.int32),
          pltpu.VMEM((CHUNK,), jnp.float32),
          pltpu.VMEM_SHARED((n_pad,), jnp.float32),
          pltpu.VMEM_SHARED((n_pad,), jnp.float32),
          pltpu.SemaphoreType.DMA((DDEPTH + 1,)),
          pltpu.SemaphoreType.DMA((DDEPTH + 1,)),
      ],
  )
  def k(ei_hbm, z_hbm, deg_hbm, idx_v, ones_v, dego_sh, degi_sh, osem, isem):
    cid = lax.axis_index("c")
    sid = lax.axis_index("s")
    w = cid * NS + sid
    pltpu.sync_copy(ei_hbm.at[0, w], idx_v.at[0])
    pltpu.sync_copy(ei_hbm.at[1, w], idx_v.at[1])
    for t in range(CHUNK // 16):
      ones_v[pl.ds(t * 16, 16)] = jnp.ones((16,), jnp.float32)
    slab = pl.ds(sid * slab_n, slab_n)
    pltpu.sync_copy(z_hbm.at[slab], dego_sh.at[slab])
    pltpu.sync_copy(z_hbm.at[slab], degi_sh.at[slab])
    plsc.subcore_barrier()

    def fire(c):
      cb = lax.rem(c, DDEPTH + 1)
      pltpu.async_copy(ones_v, dego_sh.at[idx_v.at[0, c]], osem.at[cb],
                       add=True)
      pltpu.async_copy(ones_v, degi_sh.at[idx_v.at[1, c]], isem.at[cb],
                       add=True)

    def drain(c):
      cb = lax.rem(c, DDEPTH + 1)
      pltpu.make_async_copy(
          ones_v, dego_sh.at[idx_v.at[0, c]], osem.at[cb]).wait()
      pltpu.make_async_copy(
          ones_v, degi_sh.at[idx_v.at[1, c]], isem.at[cb]).wait()

    for c in range(DDEPTH):
      fire(c)

    def body(j, carry):
      @pl.when(j + DDEPTH < nchunk)
      def _():
        fire(j + DDEPTH)

      drain(j)
      return carry

    lax.fori_loop(0, nchunk, body, 0)
    plsc.subcore_barrier()
    pltpu.sync_copy(dego_sh.at[slab], deg_hbm.at[cid, 0, slab])
    pltpu.sync_copy(degi_sh.at[slab], deg_hbm.at[cid, 1, slab])

  return k(ei, zeros1)


def _sc_aggregate(ei, h_scaled, zeros2, n_pad, nchunk, d):
  slab_n = n_pad // NS

  @functools.partial(
      pl.kernel,
      out_type=jax.ShapeDtypeStruct((NC, n_pad, d), jnp.float32),
      mesh=_mesh(),
      scratch_types=[
          pltpu.VMEM((nchunk, CHUNK), jnp.int32),
          pltpu.VMEM((ISN, CHUNK), jnp.int32),
          pltpu.VMEM((NBUF, CHUNK, d), jnp.float32),
          pltpu.VMEM_SHARED((n_pad, d), jnp.float32),
          pltpu.SemaphoreType.DMA((NBUF,)),
          pltpu.SemaphoreType.DMA((NBUF,)),
          pltpu.SemaphoreType.DMA((ISN,)),
      ],
  )
  def k(ei_hbm, h_hbm, z_hbm, agg_hbm,
        dst_v, src_v, rows_v, agg_sh, gsem, ssem, xsem):
    cid = lax.axis_index("c")
    sid = lax.axis_index("s")
    w = cid * NS + sid
    pltpu.sync_copy(ei_hbm.at[1, w], dst_v)
    slab = pl.ds(sid * slab_n, slab_n)
    pltpu.sync_copy(z_hbm.at[slab], agg_sh.at[slab])
    plsc.subcore_barrier()

    def fire_idx(r, rb):
      pltpu.async_copy(ei_hbm.at[0, w, r], src_v.at[rb], xsem.at[rb])

    def fire_gather(r, rb, bb):
      pltpu.make_async_copy(
          ei_hbm.at[0, w, r], src_v.at[rb], xsem.at[rb]).wait()
      pltpu.async_copy(h_hbm.at[src_v.at[rb]], rows_v.at[bb], gsem.at[bb])

    for r in range(min(DDEPTH, nchunk)):
      fire_idx(r, r)
    for bi in range(min(NBUF, nchunk)):
      fire_gather(bi, bi, bi)

    def body(j, carry):
      jb = lax.rem(j, NBUF)

      @pl.when(j + DDEPTH < nchunk)
      def _():
        fire_idx(j + DDEPTH, lax.rem(j + DDEPTH, ISN))

      pltpu.make_async_copy(
          h_hbm.at[src_v.at[lax.rem(j, ISN)]], rows_v.at[jb],
          gsem.at[jb]).wait()
      pltpu.async_copy(
          rows_v.at[jb], agg_sh.at[dst_v.at[j]], ssem.at[jb], add=True)

      @pl.when(j >= 1)
      def _():
        jp = j - 1
        pb = lax.rem(jp, NBUF)
        pltpu.make_async_copy(
            rows_v.at[pb], agg_sh.at[dst_v.at[jp]], ssem.at[pb]).wait()

        @pl.when(jp + NBUF < nchunk)
        def _():
          jn = jp + NBUF
          fire_gather(jn, lax.rem(jn, ISN), pb)

      return carry

    lax.fori_loop(0, nchunk, body, 0)
    pltpu.make_async_copy(
        rows_v.at[(nchunk - 1) % NBUF], agg_sh.at[dst_v.at[nchunk - 1]],
        ssem.at[(nchunk - 1) % NBUF]).wait()
    plsc.subcore_barrier()
    pltpu.sync_copy(agg_sh.at[slab], agg_hbm.at[cid, slab])

  return k(ei, h_scaled, zeros2)


def _tc_matmul_scale(x_pad, w, degp, n_pad, d):
  grid = n_pad // BLK

  def body(x_ref, w_ref, deg_ref, o_ref):
    deg = deg_ref[0, 0] + deg_ref[1, 0]
    norm = lax.rsqrt(jnp.maximum(deg, 1.0))
    h = jnp.dot(x_ref[...], w_ref[...], preferred_element_type=jnp.float32)
    o_ref[...] = h * norm

  return pl.pallas_call(
      body,
      grid=(grid,),
      in_specs=[
          pl.BlockSpec((BLK, d), lambda i: (i, 0)),
          pl.BlockSpec((d, d), lambda i: (0, 0)),
          pl.BlockSpec((NC, 2, BLK, 1), lambda i: (0, 0, i, 0)),
      ],
      out_specs=pl.BlockSpec((BLK, d), lambda i: (i, 0)),
      out_shape=jax.ShapeDtypeStruct((n_pad, d), jnp.float32),
  )(x_pad, w, degp)


def _tc_finalize(aggp, degp, b2, n_pad, d):
  grid = n_pad // BLK

  def body(agg_ref, deg_ref, b_ref, act_ref, clone_ref):
    agg = agg_ref[0] + agg_ref[1]
    deg = deg_ref[0, 1] + deg_ref[1, 1]
    norm = lax.rsqrt(jnp.maximum(deg, 1.0))
    out = agg * norm + b_ref[...]
    act = jnp.maximum(out, 0.0)
    act_ref[...] = act
    clone_ref[...] = jnp.where(act >= 0.5, 1.0, 0.0).astype(jnp.float32)

  return pl.pallas_call(
      body,
      grid=(grid,),
      in_specs=[
          pl.BlockSpec((NC, BLK, d), lambda i: (0, i, 0)),
          pl.BlockSpec((NC, 2, BLK, 1), lambda i: (0, 0, i, 0)),
          pl.BlockSpec((1, d), lambda i: (0, 0)),
      ],
      out_specs=[
          pl.BlockSpec((BLK, d), lambda i: (i, 0)),
          pl.BlockSpec((BLK, d), lambda i: (i, 0)),
      ],
      out_shape=[
          jax.ShapeDtypeStruct((n_pad, d), jnp.float32),
          jax.ShapeDtypeStruct((n_pad, d), jnp.float32),
      ],
  )(aggp, degp, b2)


def kernel(in_feat, edge_index, W, b):
  n, d = in_feat.shape
  e = edge_index.shape[1]
  n_pad = ((n + BLK - 1) // BLK) * BLK
  nchunk = -(-e // (NW * CHUNK))
  pad_len = NW * CHUNK * nchunk - e
  pad_vals = n + (jnp.arange(pad_len, dtype=jnp.int32) % (n_pad - n))
  ei = jnp.concatenate(
      [edge_index.astype(jnp.int32),
       jnp.broadcast_to(pad_vals, (2, pad_len))],
      axis=1).reshape(2, NW, nchunk, CHUNK)

  zeros1 = jnp.zeros((n_pad,), jnp.float32)
  zeros2 = jnp.zeros((n_pad, d), jnp.float32)
  x_pad = jnp.pad(in_feat, ((0, n_pad - n), (0, 0)))

  degp = _sc_degrees(ei, zeros1, n_pad, nchunk)
  degp4 = degp.reshape(NC, 2, n_pad, 1)
  h_scaled = _tc_matmul_scale(x_pad, W, degp4, n_pad, d)
  aggp = _sc_aggregate(ei, h_scaled, zeros2, n_pad, nchunk, d)
  h_act, h_clone = _tc_finalize(aggp, degp4, b.reshape(1, d), n_pad, d)
  return (h_act[:n], h_clone[:n])
